# Initial kernel scaffold; baseline (speedup 1.0000x reference)
import jax
import jax.numpy as jnp
from jax.experimental import pallas as pl


def kernel(x, edge_index, batch, W1, b1, bn1_g, bn1_b, convW, convB, bnc_g, bnc_b, mlpW, mlpB, bnm_g, bnm_b, outW, outb):
    raise NotImplementedError("write your pallas kernel here")


# speedup vs baseline: 11.7745x; 11.7745x over previous
import functools
import math

import jax
import jax.numpy as jnp
from jax import lax
from jax.experimental import pallas as pl
from jax.experimental.pallas import tpu as pltpu
from jax.experimental.pallas import tpu_sc as plsc

N = 10000
E = 320000
F_IN = 128
DIM = 64
NG = 64
NHID = 3
EPS = 1e-5

NC = 2
NS = 16
NW = NC * NS
L = 16

NP = 10240
EP = 327680
CK = 128
EW = EP // NW
NCK = EW // CK
RPW = NP // NS
PAD_ROW = 10008

BLK = 1024
G = NP // BLK

CINV = 1.0 / math.sqrt(1.0 + EPS)

_mesh = plsc.VectorSubcoreMesh(core_axis_name="c", subcore_axis_name="s")


@functools.partial(
    pl.kernel,
    out_type=jax.ShapeDtypeStruct((NC, NP, L), jnp.float32),
    mesh=_mesh,
    scratch_types=[
        pltpu.VMEM_SHARED((NP, L), jnp.float32),
        pltpu.VMEM((NCK, CK), jnp.int32),
        pltpu.VMEM((CK, L), jnp.float32),
    ],
    compiler_params=pltpu.CompilerParams(needs_layout_passes=False,
                                         use_tc_tiling_on_sc=False),
)
def _deg_kernel(dst_hbm, zeros_hbm, deg_out, acc_sh, dstb, onesb):
    c = lax.axis_index("c")
    s = lax.axis_index("s")
    wid = c * NS + s

    ones = jnp.ones((L,), jnp.float32)

    def oloop(r, carry):
        onesb[r, :] = ones
        return carry

    lax.fori_loop(0, CK, oloop, 0)

    pltpu.sync_copy(zeros_hbm.at[pl.ds(s * RPW, RPW), :],
                    acc_sh.at[pl.ds(s * RPW, RPW), :])
    pltpu.sync_copy(dst_hbm.at[pl.ds(wid * NCK, NCK), :], dstb)
    plsc.subcore_barrier()

    def eloop(j, carry):
        pltpu.sync_copy(onesb, acc_sh.at[dstb.at[j]], add=True)
        return carry

    lax.fori_loop(0, NCK, eloop, 0)

    plsc.subcore_barrier()
    pltpu.sync_copy(acc_sh.at[pl.ds(s * RPW, RPW), :],
                    deg_out.at[c, pl.ds(s * RPW, RPW), :])


@functools.partial(
    pl.kernel,
    out_type=jax.ShapeDtypeStruct((NC, NP, DIM), jnp.float32),
    mesh=_mesh,
    scratch_types=[
        pltpu.VMEM_SHARED((NP, DIM), jnp.float32),
        pltpu.VMEM((NCK, CK), jnp.int32),
        pltpu.VMEM((NCK, CK), jnp.int32),
        pltpu.VMEM((CK, DIM), jnp.float32),
        pltpu.SemaphoreType.DMA,
    ],
    compiler_params=pltpu.CompilerParams(needs_layout_passes=False,
                                         use_tc_tiling_on_sc=False),
)
def _spmm_kernel(z_hbm, src_hbm, dst_hbm, zeros_hbm, w_out,
                 acc_sh, srcb, dstb, rows, gsem):
    c = lax.axis_index("c")
    s = lax.axis_index("s")
    wid = c * NS + s

    pltpu.sync_copy(zeros_hbm.at[pl.ds(s * RPW, RPW), :],
                    acc_sh.at[pl.ds(s * RPW, RPW), :])
    pltpu.sync_copy(src_hbm.at[pl.ds(wid * NCK, NCK), :], srcb)
    pltpu.sync_copy(dst_hbm.at[pl.ds(wid * NCK, NCK), :], dstb)
    plsc.subcore_barrier()

    def body(j, carry):
        pltpu.async_copy(z_hbm.at[srcb.at[j]], rows, gsem).wait()
        pltpu.sync_copy(rows, acc_sh.at[dstb.at[j]], add=True)
        return carry

    lax.fori_loop(0, NCK, body, 0)

    plsc.subcore_barrier()
    pltpu.sync_copy(acc_sh.at[pl.ds(s * RPW, RPW), :],
                    w_out.at[c, pl.ds(s * RPW, RPW), :])


def _stage_a_body(x_ref, w1_ref, degp_ref, z_ref, dinv_ref):
    deg = degp_ref[0, :, 0:1] + degp_ref[1, :, 0:1] + 1.0
    dinv = lax.rsqrt(deg)
    h = jnp.dot(x_ref[...], w1_ref[...],
                preferred_element_type=jnp.float32,
                precision=lax.Precision.HIGHEST)
    z_ref[...] = h * dinv
    dinv_ref[...] = dinv


def _stage_conv_body(relu, w_ref, z_ref, dinv_ref, b_ref, g_ref, bb_ref,
                     wn_ref, zn_ref):
    dinv = dinv_ref[...]
    h = (w_ref[0] + w_ref[1] + z_ref[...]) * dinv + b_ref[...]
    if relu:
        h = jnp.maximum(h, 0.0)
    h = h * (g_ref[...] * CINV) + bb_ref[...]
    zn_ref[...] = jnp.dot(h, wn_ref[...],
                          preferred_element_type=jnp.float32,
                          precision=lax.Precision.HIGHEST) * dinv


def _stage_e_body(w_ref, z_ref, dinv_ref, b_ref, g_ref, bb_ref, batch_ref,
                  mlpW_ref, mlpB_ref, bnmg_ref, bnmb_ref, outW_ref, outb_ref,
                  out_ref, acc_ref):
    i = pl.program_id(0)
    h = (w_ref[0] + w_ref[1] + z_ref[...]) * dinv_ref[...] + b_ref[...]
    h = h * (g_ref[...] * CINV) + bb_ref[...]
    gids = lax.broadcasted_iota(jnp.int32, (1, NG), 1)
    onehot = (batch_ref[...] == gids).astype(jnp.float32)
    contrib = lax.dot_general(onehot, h, (((0,), (0,)), ((), ())),
                              preferred_element_type=jnp.float32,
                              precision=lax.Precision.HIGHEST)

    @pl.when(i == 0)
    def _():
        acc_ref[...] = contrib

    @pl.when(i > 0)
    def _():
        acc_ref[...] = acc_ref[...] + contrib

    @pl.when(i == G - 1)
    def _():
        p = acc_ref[...]
        for k in range(NHID):
            p = jnp.dot(p, mlpW_ref[k],
                        preferred_element_type=jnp.float32,
                        precision=lax.Precision.HIGHEST) + mlpB_ref[k]
            p = jnp.maximum(p, 0.0)
            p = p * (bnmg_ref[k] * CINV) + bnmb_ref[k]
        out_ref[...] = jnp.dot(p, outW_ref[...],
                               preferred_element_type=jnp.float32,
                               precision=lax.Precision.HIGHEST) + outb_ref[...]


def _full_spec(shape):
    return pl.BlockSpec(shape, lambda i: tuple(0 for _ in shape))


_stage_a = pl.pallas_call(
    _stage_a_body,
    grid=(G,),
    in_specs=[
        pl.BlockSpec((BLK, F_IN), lambda i: (i, 0)),
        _full_spec((F_IN, DIM)),
        pl.BlockSpec((NC, BLK, L), lambda i: (0, i, 0)),
    ],
    out_specs=[
        pl.BlockSpec((BLK, DIM), lambda i: (i, 0)),
        pl.BlockSpec((BLK, 1), lambda i: (i, 0)),
    ],
    out_shape=[
        jax.ShapeDtypeStruct((NP, DIM), jnp.float32),
        jax.ShapeDtypeStruct((NP, 1), jnp.float32),
    ],
)


def _make_stage_conv(relu):
    return pl.pallas_call(
        functools.partial(_stage_conv_body, relu),
        grid=(G,),
        in_specs=[
            pl.BlockSpec((NC, BLK, DIM), lambda i: (0, i, 0)),
            pl.BlockSpec((BLK, DIM), lambda i: (i, 0)),
            pl.BlockSpec((BLK, 1), lambda i: (i, 0)),
            _full_spec((1, DIM)),
            _full_spec((1, DIM)),
            _full_spec((1, DIM)),
            _full_spec((DIM, DIM)),
        ],
        out_specs=pl.BlockSpec((BLK, DIM), lambda i: (i, 0)),
        out_shape=jax.ShapeDtypeStruct((NP, DIM), jnp.float32),
    )


_stage_conv_relu = _make_stage_conv(True)
_stage_conv_plain = _make_stage_conv(False)

_stage_e = pl.pallas_call(
    _stage_e_body,
    grid=(G,),
    in_specs=[
        pl.BlockSpec((NC, BLK, DIM), lambda i: (0, i, 0)),
        pl.BlockSpec((BLK, DIM), lambda i: (i, 0)),
        pl.BlockSpec((BLK, 1), lambda i: (i, 0)),
        _full_spec((1, DIM)),
        _full_spec((1, DIM)),
        _full_spec((1, DIM)),
        pl.BlockSpec((BLK, 1), lambda i: (i, 0)),
        _full_spec((NHID, DIM, DIM)),
        _full_spec((NHID, DIM)),
        _full_spec((NHID, DIM)),
        _full_spec((NHID, DIM)),
        _full_spec((DIM, 1)),
        _full_spec((1, 1)),
    ],
    out_specs=pl.BlockSpec((NG, 1), lambda i: (0, 0)),
    out_shape=jax.ShapeDtypeStruct((NG, 1), jnp.float32),
    scratch_shapes=[pltpu.VMEM((NG, DIM), jnp.float32)],
)


def kernel(x, edge_index, batch, W1, b1, bn1_g, bn1_b, convW, convB,
           bnc_g, bnc_b, mlpW, mlpB, bnm_g, bnm_b, outW, outb):
    src = edge_index[0]
    dst = edge_index[1]
    pad = jnp.full((EP - E,), PAD_ROW, jnp.int32)
    src2d = jnp.concatenate([src, pad]).reshape(EP // CK, CK)
    dst2d = jnp.concatenate([dst, pad]).reshape(EP // CK, CK)
    xp = jnp.pad(x, ((0, NP - N), (0, 0)))
    batchp = jnp.concatenate(
        [batch, jnp.full((NP - N,), NG, jnp.int32)]).reshape(NP, 1)
    zeros_nd = jnp.zeros((NP, DIM), jnp.float32)
    zeros_nl = jnp.zeros((NP, L), jnp.float32)

    b1r = b1.reshape(1, DIM)
    g1r = bn1_g.reshape(1, DIM)
    bb1r = bn1_b.reshape(1, DIM)
    outbr = outb.reshape(1, 1)

    deg_parts = _deg_kernel(dst2d, zeros_nl)
    z, dinv = _stage_a(xp, W1, deg_parts)

    stage_params = [(b1r, g1r, bb1r, True)] + [
        (convB[i].reshape(1, DIM), bnc_g[i].reshape(1, DIM),
         bnc_b[i].reshape(1, DIM), False)
        for i in range(NHID)
    ]

    for li in range(NHID):
        w = _spmm_kernel(z, src2d, dst2d, zeros_nd)
        br, gr, bbr, relu = stage_params[li]
        stage = _stage_conv_relu if relu else _stage_conv_plain
        z = stage(w, z, dinv, br, gr, bbr, convW[li])

    w = _spmm_kernel(z, src2d, dst2d, zeros_nd)
    br, gr, bbr, _ = stage_params[NHID]
    out = _stage_e(w, z, dinv, br, gr, bbr, batchp,
                   mlpW, mlpB, bnm_g, bnm_b, outW, outbr)
    return out


# speedup vs baseline: 14.0937x; 1.1970x over previous
import functools
import math

import jax
import jax.numpy as jnp
from jax import lax
from jax.experimental import pallas as pl
from jax.experimental.pallas import tpu as pltpu
from jax.experimental.pallas import tpu_sc as plsc

N = 10000
E = 320000
F_IN = 128
DIM = 64
NG = 64
NHID = 3
EPS = 1e-5

NC = 2
NS = 16
NW = NC * NS
L = 16

NP = 10240
EP = 327680
CK = 128
EW = EP // NW
NCK = EW // CK
RPW = NP // NS
PAD_ROW = 10008

BLK = 1024
G = NP // BLK

CINV = 1.0 / math.sqrt(1.0 + EPS)

_mesh = plsc.VectorSubcoreMesh(core_axis_name="c", subcore_axis_name="s")


@functools.partial(
    pl.kernel,
    out_type=jax.ShapeDtypeStruct((NC, NP, L), jnp.float32),
    mesh=_mesh,
    scratch_types=[
        pltpu.VMEM_SHARED((NP, L), jnp.float32),
        pltpu.VMEM((NCK, CK), jnp.int32),
        pltpu.VMEM((CK, L), jnp.float32),
    ],
    compiler_params=pltpu.CompilerParams(needs_layout_passes=False,
                                         use_tc_tiling_on_sc=False),
)
def _deg_kernel(dst_hbm, zeros_hbm, deg_out, acc_sh, dstb, onesb):
    c = lax.axis_index("c")
    s = lax.axis_index("s")
    wid = c * NS + s

    ones = jnp.ones((L,), jnp.float32)

    def oloop(r, carry):
        onesb[r, :] = ones
        return carry

    lax.fori_loop(0, CK, oloop, 0)

    pltpu.sync_copy(zeros_hbm.at[pl.ds(s * RPW, RPW), :],
                    acc_sh.at[pl.ds(s * RPW, RPW), :])
    pltpu.sync_copy(dst_hbm.at[pl.ds(wid * NCK, NCK), :], dstb)
    plsc.subcore_barrier()

    def eloop(j, carry):
        pltpu.sync_copy(onesb, acc_sh.at[dstb.at[j]], add=True)
        return carry

    lax.fori_loop(0, NCK, eloop, 0)

    plsc.subcore_barrier()
    pltpu.sync_copy(acc_sh.at[pl.ds(s * RPW, RPW), :],
                    deg_out.at[c, pl.ds(s * RPW, RPW), :])


NBUF = 8
NGRP = NCK // NBUF


@functools.partial(
    pl.kernel,
    out_type=jax.ShapeDtypeStruct((NC, NP, DIM), jnp.float32),
    mesh=_mesh,
    scratch_types=[
        pltpu.VMEM_SHARED((NP, DIM), jnp.float32),
        pltpu.VMEM((NCK, CK), jnp.int32),
        pltpu.VMEM((NCK, CK), jnp.int32),
    ] + [pltpu.VMEM((CK, DIM), jnp.float32) for _ in range(NBUF)]
      + [pltpu.SemaphoreType.DMA for _ in range(2 * NBUF)],
    compiler_params=pltpu.CompilerParams(needs_layout_passes=False,
                                         use_tc_tiling_on_sc=False),
)
def _spmm_kernel(z_hbm, src_hbm, dst_hbm, zeros_hbm, w_out,
                 acc_sh, srcb, dstb, *bufs_sems):
    rows = bufs_sems[:NBUF]
    gsem = bufs_sems[NBUF:2 * NBUF]
    ssem = bufs_sems[2 * NBUF:]
    c = lax.axis_index("c")
    s = lax.axis_index("s")
    wid = c * NS + s

    pltpu.sync_copy(zeros_hbm.at[pl.ds(s * RPW, RPW), :],
                    acc_sh.at[pl.ds(s * RPW, RPW), :])
    pltpu.sync_copy(src_hbm.at[pl.ds(wid * NCK, NCK), :], srcb)
    pltpu.sync_copy(dst_hbm.at[pl.ds(wid * NCK, NCK), :], dstb)
    plsc.subcore_barrier()

    def _gather(j, b):
        return pltpu.async_copy(z_hbm.at[srcb.at[j]], rows[b], gsem[b])

    def _scatter(j, b):
        return pltpu.async_copy(rows[b], acc_sh.at[dstb.at[j]], ssem[b],
                                add=True)

    def _gather_wait(j, b):
        pltpu.make_async_copy(z_hbm.at[srcb.at[j]], rows[b], gsem[b]).wait()

    def _scatter_wait(j, b):
        pltpu.make_async_copy(rows[b], acc_sh.at[dstb.at[j]],
                              ssem[b]).wait()

    for b in range(NBUF):
        _gather(b, b)

    def grp_body(grp, carry):
        j0 = grp * NBUF
        for b in range(NBUF):
            _gather_wait(j0 + b, b)
            _scatter(j0 + b, b)
        for b in range(NBUF):
            _scatter_wait(j0 + b, b)
            _gather(j0 + NBUF + b, b)
        return carry

    lax.fori_loop(0, NGRP - 1, grp_body, 0)

    jl = (NGRP - 1) * NBUF
    for b in range(NBUF):
        _gather_wait(jl + b, b)
        _scatter(jl + b, b)
    for b in range(NBUF):
        _scatter_wait(jl + b, b)

    plsc.subcore_barrier()
    pltpu.sync_copy(acc_sh.at[pl.ds(s * RPW, RPW), :],
                    w_out.at[c, pl.ds(s * RPW, RPW), :])


def _stage_a_body(x_ref, w1_ref, degp_ref, z_ref, dinv_ref):
    deg = degp_ref[0, :, 0:1] + degp_ref[1, :, 0:1] + 1.0
    dinv = lax.rsqrt(deg)
    h = jnp.dot(x_ref[...], w1_ref[...],
                preferred_element_type=jnp.float32,
                precision=lax.Precision.HIGHEST)
    z_ref[...] = h * dinv
    dinv_ref[...] = dinv


def _stage_conv_body(relu, w_ref, z_ref, dinv_ref, b_ref, g_ref, bb_ref,
                     wn_ref, zn_ref):
    dinv = dinv_ref[...]
    h = (w_ref[0] + w_ref[1] + z_ref[...]) * dinv + b_ref[...]
    if relu:
        h = jnp.maximum(h, 0.0)
    h = h * (g_ref[...] * CINV) + bb_ref[...]
    zn_ref[...] = jnp.dot(h, wn_ref[...],
                          preferred_element_type=jnp.float32,
                          precision=lax.Precision.HIGHEST) * dinv


def _stage_e_body(w_ref, z_ref, dinv_ref, b_ref, g_ref, bb_ref, batch_ref,
                  mlpW_ref, mlpB_ref, bnmg_ref, bnmb_ref, outW_ref, outb_ref,
                  out_ref, acc_ref):
    i = pl.program_id(0)
    h = (w_ref[0] + w_ref[1] + z_ref[...]) * dinv_ref[...] + b_ref[...]
    h = h * (g_ref[...] * CINV) + bb_ref[...]
    gids = lax.broadcasted_iota(jnp.int32, (1, NG), 1)
    onehot = (batch_ref[...] == gids).astype(jnp.float32)
    contrib = lax.dot_general(onehot, h, (((0,), (0,)), ((), ())),
                              preferred_element_type=jnp.float32,
                              precision=lax.Precision.HIGHEST)

    @pl.when(i == 0)
    def _():
        acc_ref[...] = contrib

    @pl.when(i > 0)
    def _():
        acc_ref[...] = acc_ref[...] + contrib

    @pl.when(i == G - 1)
    def _():
        p = acc_ref[...]
        for k in range(NHID):
            p = jnp.dot(p, mlpW_ref[k],
                        preferred_element_type=jnp.float32,
                        precision=lax.Precision.HIGHEST) + mlpB_ref[k]
            p = jnp.maximum(p, 0.0)
            p = p * (bnmg_ref[k] * CINV) + bnmb_ref[k]
        out_ref[...] = jnp.dot(p, outW_ref[...],
                               preferred_element_type=jnp.float32,
                               precision=lax.Precision.HIGHEST) + outb_ref[...]


def _full_spec(shape):
    return pl.BlockSpec(shape, lambda i: tuple(0 for _ in shape))


_stage_a = pl.pallas_call(
    _stage_a_body,
    grid=(G,),
    in_specs=[
        pl.BlockSpec((BLK, F_IN), lambda i: (i, 0)),
        _full_spec((F_IN, DIM)),
        pl.BlockSpec((NC, BLK, L), lambda i: (0, i, 0)),
    ],
    out_specs=[
        pl.BlockSpec((BLK, DIM), lambda i: (i, 0)),
        pl.BlockSpec((BLK, 1), lambda i: (i, 0)),
    ],
    out_shape=[
        jax.ShapeDtypeStruct((NP, DIM), jnp.float32),
        jax.ShapeDtypeStruct((NP, 1), jnp.float32),
    ],
)


def _make_stage_conv(relu):
    return pl.pallas_call(
        functools.partial(_stage_conv_body, relu),
        grid=(G,),
        in_specs=[
            pl.BlockSpec((NC, BLK, DIM), lambda i: (0, i, 0)),
            pl.BlockSpec((BLK, DIM), lambda i: (i, 0)),
            pl.BlockSpec((BLK, 1), lambda i: (i, 0)),
            _full_spec((1, DIM)),
            _full_spec((1, DIM)),
            _full_spec((1, DIM)),
            _full_spec((DIM, DIM)),
        ],
        out_specs=pl.BlockSpec((BLK, DIM), lambda i: (i, 0)),
        out_shape=jax.ShapeDtypeStruct((NP, DIM), jnp.float32),
    )


_stage_conv_relu = _make_stage_conv(True)
_stage_conv_plain = _make_stage_conv(False)

_stage_e = pl.pallas_call(
    _stage_e_body,
    grid=(G,),
    in_specs=[
        pl.BlockSpec((NC, BLK, DIM), lambda i: (0, i, 0)),
        pl.BlockSpec((BLK, DIM), lambda i: (i, 0)),
        pl.BlockSpec((BLK, 1), lambda i: (i, 0)),
        _full_spec((1, DIM)),
        _full_spec((1, DIM)),
        _full_spec((1, DIM)),
        pl.BlockSpec((BLK, 1), lambda i: (i, 0)),
        _full_spec((NHID, DIM, DIM)),
        _full_spec((NHID, DIM)),
        _full_spec((NHID, DIM)),
        _full_spec((NHID, DIM)),
        _full_spec((DIM, 1)),
        _full_spec((1, 1)),
    ],
    out_specs=pl.BlockSpec((NG, 1), lambda i: (0, 0)),
    out_shape=jax.ShapeDtypeStruct((NG, 1), jnp.float32),
    scratch_shapes=[pltpu.VMEM((NG, DIM), jnp.float32)],
)


def kernel(x, edge_index, batch, W1, b1, bn1_g, bn1_b, convW, convB,
           bnc_g, bnc_b, mlpW, mlpB, bnm_g, bnm_b, outW, outb):
    src = edge_index[0]
    dst = edge_index[1]
    pad = jnp.full((EP - E,), PAD_ROW, jnp.int32)
    src2d = jnp.concatenate([src, pad]).reshape(EP // CK, CK)
    dst2d = jnp.concatenate([dst, pad]).reshape(EP // CK, CK)
    xp = jnp.pad(x, ((0, NP - N), (0, 0)))
    batchp = jnp.concatenate(
        [batch, jnp.full((NP - N,), NG, jnp.int32)]).reshape(NP, 1)
    zeros_nd = jnp.zeros((NP, DIM), jnp.float32)
    zeros_nl = jnp.zeros((NP, L), jnp.float32)

    b1r = b1.reshape(1, DIM)
    g1r = bn1_g.reshape(1, DIM)
    bb1r = bn1_b.reshape(1, DIM)
    outbr = outb.reshape(1, 1)

    deg_parts = _deg_kernel(dst2d, zeros_nl)
    z, dinv = _stage_a(xp, W1, deg_parts)

    stage_params = [(b1r, g1r, bb1r, True)] + [
        (convB[i].reshape(1, DIM), bnc_g[i].reshape(1, DIM),
         bnc_b[i].reshape(1, DIM), False)
        for i in range(NHID)
    ]

    for li in range(NHID):
        w = _spmm_kernel(z, src2d, dst2d, zeros_nd)
        br, gr, bbr, relu = stage_params[li]
        stage = _stage_conv_relu if relu else _stage_conv_plain
        z = stage(w, z, dinv, br, gr, bbr, convW[li])

    w = _spmm_kernel(z, src2d, dst2d, zeros_nd)
    br, gr, bbr, _ = stage_params[NHID]
    out = _stage_e(w, z, dinv, br, gr, bbr, batchp,
                   mlpW, mlpB, bnm_g, bnm_b, outW, outbr)
    return out


# speedup vs baseline: 35.0685x; 2.4882x over previous
import functools
import math

import jax
import jax.numpy as jnp
from jax import lax
from jax.experimental import pallas as pl
from jax.experimental.pallas import tpu as pltpu
from jax.experimental.pallas import tpu_sc as plsc

N = 10000
E = 320000
F_IN = 128
DIM = 64
NG = 64
NHID = 3
EPS = 1e-5

NC = 2
NS = 16
NW = NC * NS
L = 16

NP = 10240
EP = 327680
CK = 128
EW = EP // NW
NCK = EW // CK
RPW = NP // NS
PAD_ROW = 10008

BLK = 1024
G = NP // BLK

CINV = 1.0 / math.sqrt(1.0 + EPS)

_mesh = plsc.VectorSubcoreMesh(core_axis_name="c", subcore_axis_name="s")


@functools.partial(
    pl.kernel,
    out_type=jax.ShapeDtypeStruct((NC, NP, L), jnp.float32),
    mesh=_mesh,
    scratch_types=[
        pltpu.VMEM_SHARED((NP, L), jnp.float32),
        pltpu.VMEM((NCK, CK), jnp.int32),
        pltpu.VMEM((CK, L), jnp.float32),
    ],
    compiler_params=pltpu.CompilerParams(needs_layout_passes=False,
                                         use_tc_tiling_on_sc=False),
)
def _deg_kernel(dst_hbm, zeros_hbm, deg_out, acc_sh, dstb, onesb):
    c = lax.axis_index("c")
    s = lax.axis_index("s")
    wid = c * NS + s

    ones = jnp.ones((L,), jnp.float32)

    def oloop(r, carry):
        onesb[r, :] = ones
        return carry

    lax.fori_loop(0, CK, oloop, 0)

    pltpu.sync_copy(zeros_hbm.at[pl.ds(s * RPW, RPW), :],
                    acc_sh.at[pl.ds(s * RPW, RPW), :])
    pltpu.sync_copy(dst_hbm.at[pl.ds(wid * NCK, NCK), :], dstb)
    plsc.subcore_barrier()

    def eloop(j, carry):
        pltpu.sync_copy(onesb, acc_sh.at[dstb.at[j]], add=True)
        return carry

    lax.fori_loop(0, NCK, eloop, 0)

    plsc.subcore_barrier()
    pltpu.sync_copy(acc_sh.at[pl.ds(s * RPW, RPW), :],
                    deg_out.at[c, pl.ds(s * RPW, RPW), :])


NBUF = 8
NGRP = NCK // NBUF


@functools.partial(
    pl.kernel,
    out_type=jax.ShapeDtypeStruct((NC, NP, DIM), jnp.float32),
    mesh=_mesh,
    scratch_types=[
        pltpu.VMEM_SHARED((NP, DIM), jnp.float32),
        pltpu.VMEM((NCK, CK), jnp.int32),
        pltpu.VMEM((NCK, CK), jnp.int32),
    ] + [pltpu.VMEM((CK, DIM), jnp.float32) for _ in range(NBUF)]
      + [pltpu.SemaphoreType.DMA for _ in range(2 * NBUF)],
    compiler_params=pltpu.CompilerParams(needs_layout_passes=False,
                                         use_tc_tiling_on_sc=False),
)
def _spmm_kernel(z_hbm, src_hbm, dst_hbm, zeros_hbm, w_out,
                 acc_sh, srcb, dstb, *bufs_sems):
    rows = bufs_sems[:NBUF]
    gsem = bufs_sems[NBUF:2 * NBUF]
    ssem = bufs_sems[2 * NBUF:]
    c = lax.axis_index("c")
    s = lax.axis_index("s")
    wid = c * NS + s

    pltpu.sync_copy(zeros_hbm.at[pl.ds(s * RPW, RPW), :],
                    acc_sh.at[pl.ds(s * RPW, RPW), :])
    pltpu.sync_copy(src_hbm.at[pl.ds(wid * NCK, NCK), :], srcb)
    pltpu.sync_copy(dst_hbm.at[pl.ds(wid * NCK, NCK), :], dstb)
    plsc.subcore_barrier()

    def _gather(j, b):
        return pltpu.async_copy(z_hbm.at[srcb.at[j]], rows[b], gsem[b])

    def _scatter(j, b):
        return pltpu.async_copy(rows[b], acc_sh.at[dstb.at[j]], ssem[b],
                                add=True)

    def _gather_wait(j, b):
        pltpu.make_async_copy(z_hbm.at[srcb.at[j]], rows[b], gsem[b]).wait()

    def _scatter_wait(j, b):
        pltpu.make_async_copy(rows[b], acc_sh.at[dstb.at[j]],
                              ssem[b]).wait()

    for b in range(NBUF):
        _gather(b, b)

    def grp_body(grp, carry):
        j0 = grp * NBUF
        for b in range(NBUF):
            _gather_wait(j0 + b, b)
            _scatter(j0 + b, b)
        for b in range(NBUF):
            _scatter_wait(j0 + b, b)
            _gather(j0 + NBUF + b, b)
        return carry

    lax.fori_loop(0, NGRP - 1, grp_body, 0)

    jl = (NGRP - 1) * NBUF
    for b in range(NBUF):
        _gather_wait(jl + b, b)
        _scatter(jl + b, b)
    for b in range(NBUF):
        _scatter_wait(jl + b, b)

    plsc.subcore_barrier()
    pltpu.sync_copy(acc_sh.at[pl.ds(s * RPW, RPW), :],
                    w_out.at[c, pl.ds(s * RPW, RPW), :])


def _stage_a_body(x_ref, w1_ref, degp_ref, z_ref, dinv_ref):
    deg = degp_ref[0, :, 0:1] + degp_ref[1, :, 0:1] + 1.0
    dinv = lax.rsqrt(deg)
    h = jnp.dot(x_ref[...], w1_ref[...],
                preferred_element_type=jnp.float32,
                precision=lax.Precision.HIGHEST)
    z_ref[...] = h * dinv
    dinv_ref[...] = dinv


def _stage_conv_body(relu, w_ref, z_ref, dinv_ref, b_ref, g_ref, bb_ref,
                     wn_ref, zn_ref):
    dinv = dinv_ref[...]
    h = (w_ref[0] + w_ref[1] + z_ref[...]) * dinv + b_ref[...]
    if relu:
        h = jnp.maximum(h, 0.0)
    h = h * (g_ref[...] * CINV) + bb_ref[...]
    zn_ref[...] = jnp.dot(h, wn_ref[...],
                          preferred_element_type=jnp.float32,
                          precision=lax.Precision.HIGHEST) * dinv


def _stage_e_body(w_ref, z_ref, dinv_ref, b_ref, g_ref, bb_ref, batch_ref,
                  mlpW_ref, mlpB_ref, bnmg_ref, bnmb_ref, outW_ref, outb_ref,
                  out_ref, acc_ref):
    i = pl.program_id(0)
    h = (w_ref[0] + w_ref[1] + z_ref[...]) * dinv_ref[...] + b_ref[...]
    h = h * (g_ref[...] * CINV) + bb_ref[...]
    gids = lax.broadcasted_iota(jnp.int32, (1, NG), 1)
    onehot = (batch_ref[...] == gids).astype(jnp.float32)
    contrib = lax.dot_general(onehot, h, (((0,), (0,)), ((), ())),
                              preferred_element_type=jnp.float32,
                              precision=lax.Precision.HIGHEST)

    @pl.when(i == 0)
    def _():
        acc_ref[...] = contrib

    @pl.when(i > 0)
    def _():
        acc_ref[...] = acc_ref[...] + contrib

    @pl.when(i == G - 1)
    def _():
        p = acc_ref[...]
        for k in range(NHID):
            p = jnp.dot(p, mlpW_ref[k],
                        preferred_element_type=jnp.float32,
                        precision=lax.Precision.HIGHEST) + mlpB_ref[k]
            p = jnp.maximum(p, 0.0)
            p = p * (bnmg_ref[k] * CINV) + bnmb_ref[k]
        out_ref[...] = jnp.dot(p, outW_ref[...],
                               preferred_element_type=jnp.float32,
                               precision=lax.Precision.HIGHEST) + outb_ref[...]


def _full_spec(shape):
    return pl.BlockSpec(shape, lambda i: tuple(0 for _ in shape))


_stage_a = pl.pallas_call(
    _stage_a_body,
    grid=(G,),
    in_specs=[
        pl.BlockSpec((BLK, F_IN), lambda i: (i, 0)),
        _full_spec((F_IN, DIM)),
        pl.BlockSpec((NC, BLK, L), lambda i: (0, i, 0)),
    ],
    out_specs=[
        pl.BlockSpec((BLK, DIM), lambda i: (i, 0)),
        pl.BlockSpec((BLK, 1), lambda i: (i, 0)),
    ],
    out_shape=[
        jax.ShapeDtypeStruct((NP, DIM), jnp.float32),
        jax.ShapeDtypeStruct((NP, 1), jnp.float32),
    ],
)


def _make_stage_conv(relu):
    return pl.pallas_call(
        functools.partial(_stage_conv_body, relu),
        grid=(G,),
        in_specs=[
            pl.BlockSpec((NC, BLK, DIM), lambda i: (0, i, 0)),
            pl.BlockSpec((BLK, DIM), lambda i: (i, 0)),
            pl.BlockSpec((BLK, 1), lambda i: (i, 0)),
            _full_spec((1, DIM)),
            _full_spec((1, DIM)),
            _full_spec((1, DIM)),
            _full_spec((DIM, DIM)),
        ],
        out_specs=pl.BlockSpec((BLK, DIM), lambda i: (i, 0)),
        out_shape=jax.ShapeDtypeStruct((NP, DIM), jnp.float32),
    )


_stage_conv_relu = _make_stage_conv(True)
_stage_conv_plain = _make_stage_conv(False)

_stage_e = pl.pallas_call(
    _stage_e_body,
    grid=(G,),
    in_specs=[
        pl.BlockSpec((NC, BLK, DIM), lambda i: (0, i, 0)),
        pl.BlockSpec((BLK, DIM), lambda i: (i, 0)),
        pl.BlockSpec((BLK, 1), lambda i: (i, 0)),
        _full_spec((1, DIM)),
        _full_spec((1, DIM)),
        _full_spec((1, DIM)),
        pl.BlockSpec((BLK, 1), lambda i: (i, 0)),
        _full_spec((NHID, DIM, DIM)),
        _full_spec((NHID, DIM)),
        _full_spec((NHID, DIM)),
        _full_spec((NHID, DIM)),
        _full_spec((DIM, 1)),
        _full_spec((1, 1)),
    ],
    out_specs=pl.BlockSpec((NG, 1), lambda i: (0, 0)),
    out_shape=jax.ShapeDtypeStruct((NG, 1), jnp.float32),
    scratch_shapes=[pltpu.VMEM((NG, DIM), jnp.float32)],
)


def kernel(x, edge_index, batch, W1, b1, bn1_g, bn1_b, convW, convB,
           bnc_g, bnc_b, mlpW, mlpB, bnm_g, bnm_b, outW, outb):
    src = edge_index[0]
    dst = edge_index[1]
    pad = N + jnp.arange(EP - E, dtype=jnp.int32) % (NP - N)
    src2d = jnp.concatenate([src, pad]).reshape(EP // CK, CK)
    dst2d = jnp.concatenate([dst, pad]).reshape(EP // CK, CK)
    xp = jnp.pad(x, ((0, NP - N), (0, 0)))
    batchp = jnp.concatenate(
        [batch, jnp.full((NP - N,), NG, jnp.int32)]).reshape(NP, 1)
    zeros_nd = jnp.zeros((NP, DIM), jnp.float32)
    zeros_nl = jnp.zeros((NP, L), jnp.float32)

    b1r = b1.reshape(1, DIM)
    g1r = bn1_g.reshape(1, DIM)
    bb1r = bn1_b.reshape(1, DIM)
    outbr = outb.reshape(1, 1)

    deg_parts = _deg_kernel(dst2d, zeros_nl)
    z, dinv = _stage_a(xp, W1, deg_parts)

    stage_params = [(b1r, g1r, bb1r, True)] + [
        (convB[i].reshape(1, DIM), bnc_g[i].reshape(1, DIM),
         bnc_b[i].reshape(1, DIM), False)
        for i in range(NHID)
    ]

    for li in range(NHID):
        w = _spmm_kernel(z, src2d, dst2d, zeros_nd)
        br, gr, bbr, relu = stage_params[li]
        stage = _stage_conv_relu if relu else _stage_conv_plain
        z = stage(w, z, dinv, br, gr, bbr, convW[li])

    w = _spmm_kernel(z, src2d, dst2d, zeros_nd)
    br, gr, bbr, _ = stage_params[NHID]
    out = _stage_e(w, z, dinv, br, gr, bbr, batchp,
                   mlpW, mlpB, bnm_g, bnm_b, outW, outbr)
    return out


# speedup vs baseline: 36.6149x; 1.0441x over previous
import functools
import math

import jax
import jax.numpy as jnp
from jax import lax
from jax.experimental import pallas as pl
from jax.experimental.pallas import tpu as pltpu
from jax.experimental.pallas import tpu_sc as plsc

N = 10000
E = 320000
F_IN = 128
DIM = 64
NG = 64
NHID = 3
EPS = 1e-5

NC = 2
NS = 16
NW = NC * NS
L = 16

NP = 10240
EP = 327680
CK = 128
EW = EP // NW
NCK = EW // CK
RPW = NP // NS
PAD_ROW = 10008

BLK = 2048
G = NP // BLK

CINV = 1.0 / math.sqrt(1.0 + EPS)

_mesh = plsc.VectorSubcoreMesh(core_axis_name="c", subcore_axis_name="s")


@functools.partial(
    pl.kernel,
    out_type=jax.ShapeDtypeStruct((NC, NP, L), jnp.float32),
    mesh=_mesh,
    scratch_types=[
        pltpu.VMEM_SHARED((NP, L), jnp.float32),
        pltpu.VMEM((NCK, CK), jnp.int32),
        pltpu.VMEM((CK, L), jnp.float32),
    ],
    compiler_params=pltpu.CompilerParams(needs_layout_passes=False,
                                         use_tc_tiling_on_sc=False),
)
def _deg_kernel(dst_hbm, zeros_hbm, deg_out, acc_sh, dstb, onesb):
    c = lax.axis_index("c")
    s = lax.axis_index("s")
    wid = c * NS + s

    ones = jnp.ones((L,), jnp.float32)

    def oloop(r, carry):
        onesb[r, :] = ones
        return carry

    lax.fori_loop(0, CK, oloop, 0)

    pltpu.sync_copy(zeros_hbm.at[pl.ds(s * RPW, RPW), :],
                    acc_sh.at[pl.ds(s * RPW, RPW), :])
    pltpu.sync_copy(dst_hbm.at[pl.ds(wid * NCK, NCK), :], dstb)
    plsc.subcore_barrier()

    def eloop(j, carry):
        pltpu.sync_copy(onesb, acc_sh.at[dstb.at[j]], add=True)
        return carry

    lax.fori_loop(0, NCK, eloop, 0)

    plsc.subcore_barrier()
    pltpu.sync_copy(acc_sh.at[pl.ds(s * RPW, RPW), :],
                    deg_out.at[c, pl.ds(s * RPW, RPW), :])


NBUF = 8
NGRP = NCK // NBUF


@functools.partial(
    pl.kernel,
    out_type=jax.ShapeDtypeStruct((NC, NP, DIM), jnp.float32),
    mesh=_mesh,
    scratch_types=[
        pltpu.VMEM_SHARED((NP, DIM), jnp.float32),
        pltpu.VMEM((NCK, CK), jnp.int32),
        pltpu.VMEM((NCK, CK), jnp.int32),
    ] + [pltpu.VMEM((CK, DIM), jnp.float32) for _ in range(NBUF)]
      + [pltpu.SemaphoreType.DMA for _ in range(2 * NBUF)],
    compiler_params=pltpu.CompilerParams(needs_layout_passes=False,
                                         use_tc_tiling_on_sc=False),
)
def _spmm_kernel(z_hbm, src_hbm, dst_hbm, zeros_hbm, w_out,
                 acc_sh, srcb, dstb, *bufs_sems):
    rows = bufs_sems[:NBUF]
    gsem = bufs_sems[NBUF:2 * NBUF]
    ssem = bufs_sems[2 * NBUF:]
    c = lax.axis_index("c")
    s = lax.axis_index("s")
    wid = c * NS + s

    d0 = pltpu.async_copy(zeros_hbm.at[pl.ds(s * RPW, RPW), :],
                          acc_sh.at[pl.ds(s * RPW, RPW), :], gsem[0])
    d1 = pltpu.async_copy(src_hbm.at[pl.ds(wid * NCK, NCK), :], srcb,
                          gsem[1])
    d2 = pltpu.async_copy(dst_hbm.at[pl.ds(wid * NCK, NCK), :], dstb,
                          gsem[2])
    d0.wait()
    d1.wait()
    d2.wait()
    plsc.subcore_barrier()

    def _gather(j, b):
        return pltpu.async_copy(z_hbm.at[srcb.at[j]], rows[b], gsem[b])

    def _scatter(j, b):
        return pltpu.async_copy(rows[b], acc_sh.at[dstb.at[j]], ssem[b],
                                add=True)

    def _gather_wait(j, b):
        pltpu.make_async_copy(z_hbm.at[srcb.at[j]], rows[b], gsem[b]).wait()

    def _scatter_wait(j, b):
        pltpu.make_async_copy(rows[b], acc_sh.at[dstb.at[j]],
                              ssem[b]).wait()

    for b in range(NBUF):
        _gather(b, b)

    def grp_body(grp, carry):
        j0 = grp * NBUF
        for b in range(NBUF):
            _gather_wait(j0 + b, b)
            _scatter(j0 + b, b)
        for b in range(NBUF):
            _scatter_wait(j0 + b, b)
            _gather(j0 + NBUF + b, b)
        return carry

    lax.fori_loop(0, NGRP - 1, grp_body, 0)

    jl = (NGRP - 1) * NBUF
    for b in range(NBUF):
        _gather_wait(jl + b, b)
        _scatter(jl + b, b)
    for b in range(NBUF):
        _scatter_wait(jl + b, b)

    plsc.subcore_barrier()
    pltpu.sync_copy(acc_sh.at[pl.ds(s * RPW, RPW), :],
                    w_out.at[c, pl.ds(s * RPW, RPW), :])


def _stage_a_body(x_ref, w1_ref, degp_ref, z_ref, dinv_ref):
    deg = degp_ref[0, :, 0:1] + degp_ref[1, :, 0:1] + 1.0
    dinv = lax.rsqrt(deg)
    h = jnp.dot(x_ref[...], w1_ref[...],
                preferred_element_type=jnp.float32,
                precision=lax.Precision.HIGHEST)
    z_ref[...] = h * dinv
    dinv_ref[...] = dinv


def _stage_conv_body(relu, w_ref, z_ref, dinv_ref, b_ref, g_ref, bb_ref,
                     wn_ref, zn_ref):
    dinv = dinv_ref[...]
    h = (w_ref[0] + w_ref[1] + z_ref[...]) * dinv + b_ref[...]
    if relu:
        h = jnp.maximum(h, 0.0)
    h = h * (g_ref[...] * CINV) + bb_ref[...]
    zn_ref[...] = jnp.dot(h, wn_ref[...],
                          preferred_element_type=jnp.float32,
                          precision=lax.Precision.HIGHEST) * dinv


def _stage_e_body(w_ref, z_ref, dinv_ref, b_ref, g_ref, bb_ref, batch_ref,
                  mlpW_ref, mlpB_ref, bnmg_ref, bnmb_ref, outW_ref, outb_ref,
                  out_ref, acc_ref):
    i = pl.program_id(0)
    h = (w_ref[0] + w_ref[1] + z_ref[...]) * dinv_ref[...] + b_ref[...]
    h = h * (g_ref[...] * CINV) + bb_ref[...]
    gids = lax.broadcasted_iota(jnp.int32, (1, NG), 1)
    onehot = (batch_ref[...] == gids).astype(jnp.float32)
    contrib = lax.dot_general(onehot, h, (((0,), (0,)), ((), ())),
                              preferred_element_type=jnp.float32,
                              precision=lax.Precision.HIGHEST)

    @pl.when(i == 0)
    def _():
        acc_ref[...] = contrib

    @pl.when(i > 0)
    def _():
        acc_ref[...] = acc_ref[...] + contrib

    @pl.when(i == G - 1)
    def _():
        p = acc_ref[...]
        for k in range(NHID):
            p = jnp.dot(p, mlpW_ref[k],
                        preferred_element_type=jnp.float32,
                        precision=lax.Precision.HIGHEST) + mlpB_ref[k]
            p = jnp.maximum(p, 0.0)
            p = p * (bnmg_ref[k] * CINV) + bnmb_ref[k]
        out_ref[...] = jnp.dot(p, outW_ref[...],
                               preferred_element_type=jnp.float32,
                               precision=lax.Precision.HIGHEST) + outb_ref[...]


def _full_spec(shape):
    return pl.BlockSpec(shape, lambda i: tuple(0 for _ in shape))


_stage_a = pl.pallas_call(
    _stage_a_body,
    grid=(G,),
    in_specs=[
        pl.BlockSpec((BLK, F_IN), lambda i: (i, 0)),
        _full_spec((F_IN, DIM)),
        pl.BlockSpec((NC, BLK, L), lambda i: (0, i, 0)),
    ],
    out_specs=[
        pl.BlockSpec((BLK, DIM), lambda i: (i, 0)),
        pl.BlockSpec((BLK, 1), lambda i: (i, 0)),
    ],
    out_shape=[
        jax.ShapeDtypeStruct((NP, DIM), jnp.float32),
        jax.ShapeDtypeStruct((NP, 1), jnp.float32),
    ],
)


def _make_stage_conv(relu):
    return pl.pallas_call(
        functools.partial(_stage_conv_body, relu),
        grid=(G,),
        in_specs=[
            pl.BlockSpec((NC, BLK, DIM), lambda i: (0, i, 0)),
            pl.BlockSpec((BLK, DIM), lambda i: (i, 0)),
            pl.BlockSpec((BLK, 1), lambda i: (i, 0)),
            _full_spec((1, DIM)),
            _full_spec((1, DIM)),
            _full_spec((1, DIM)),
            _full_spec((DIM, DIM)),
        ],
        out_specs=pl.BlockSpec((BLK, DIM), lambda i: (i, 0)),
        out_shape=jax.ShapeDtypeStruct((NP, DIM), jnp.float32),
    )


_stage_conv_relu = _make_stage_conv(True)
_stage_conv_plain = _make_stage_conv(False)

_stage_e = pl.pallas_call(
    _stage_e_body,
    grid=(G,),
    in_specs=[
        pl.BlockSpec((NC, BLK, DIM), lambda i: (0, i, 0)),
        pl.BlockSpec((BLK, DIM), lambda i: (i, 0)),
        pl.BlockSpec((BLK, 1), lambda i: (i, 0)),
        _full_spec((1, DIM)),
        _full_spec((1, DIM)),
        _full_spec((1, DIM)),
        pl.BlockSpec((BLK, 1), lambda i: (i, 0)),
        _full_spec((NHID, DIM, DIM)),
        _full_spec((NHID, DIM)),
        _full_spec((NHID, DIM)),
        _full_spec((NHID, DIM)),
        _full_spec((DIM, 1)),
        _full_spec((1, 1)),
    ],
    out_specs=pl.BlockSpec((NG, 1), lambda i: (0, 0)),
    out_shape=jax.ShapeDtypeStruct((NG, 1), jnp.float32),
    scratch_shapes=[pltpu.VMEM((NG, DIM), jnp.float32)],
)


def kernel(x, edge_index, batch, W1, b1, bn1_g, bn1_b, convW, convB,
           bnc_g, bnc_b, mlpW, mlpB, bnm_g, bnm_b, outW, outb):
    src = edge_index[0]
    dst = edge_index[1]
    pad = N + jnp.arange(EP - E, dtype=jnp.int32) % (NP - N)
    src2d = jnp.concatenate([src, pad]).reshape(EP // CK, CK)
    dst2d = jnp.concatenate([dst, pad]).reshape(EP // CK, CK)
    xp = jnp.pad(x, ((0, NP - N), (0, 0)))
    batchp = jnp.concatenate(
        [batch, jnp.full((NP - N,), NG, jnp.int32)]).reshape(NP, 1)
    zeros_nd = jnp.zeros((NP, DIM), jnp.float32)
    zeros_nl = jnp.zeros((NP, L), jnp.float32)

    b1r = b1.reshape(1, DIM)
    g1r = bn1_g.reshape(1, DIM)
    bb1r = bn1_b.reshape(1, DIM)
    outbr = outb.reshape(1, 1)

    deg_parts = _deg_kernel(dst2d, zeros_nl)
    z, dinv = _stage_a(xp, W1, deg_parts)

    stage_params = [(b1r, g1r, bb1r, True)] + [
        (convB[i].reshape(1, DIM), bnc_g[i].reshape(1, DIM),
         bnc_b[i].reshape(1, DIM), False)
        for i in range(NHID)
    ]

    for li in range(NHID):
        w = _spmm_kernel(z, src2d, dst2d, zeros_nd)
        br, gr, bbr, relu = stage_params[li]
        stage = _stage_conv_relu if relu else _stage_conv_plain
        z = stage(w, z, dinv, br, gr, bbr, convW[li])

    w = _spmm_kernel(z, src2d, dst2d, zeros_nd)
    br, gr, bbr, _ = stage_params[NHID]
    out = _stage_e(w, z, dinv, br, gr, bbr, batchp,
                   mlpW, mlpB, bnm_g, bnm_b, outW, outbr)
    return out


# speedup vs baseline: 36.8681x; 1.0069x over previous
import functools
import math

import jax
import jax.numpy as jnp
from jax import lax
from jax.experimental import pallas as pl
from jax.experimental.pallas import tpu as pltpu
from jax.experimental.pallas import tpu_sc as plsc

N = 10000
E = 320000
F_IN = 128
DIM = 64
NG = 64
NHID = 3
EPS = 1e-5

NC = 2
NS = 16
NW = NC * NS
L = 16

NP = 10240
EP = 327680
CK = 128
EW = EP // NW
NCK = EW // CK
RPW = NP // NS
PAD_ROW = 10008

BLK = 2048
G = NP // BLK

CINV = 1.0 / math.sqrt(1.0 + EPS)

_mesh = plsc.VectorSubcoreMesh(core_axis_name="c", subcore_axis_name="s")


@functools.partial(
    pl.kernel,
    out_type=jax.ShapeDtypeStruct((NC, NP, L), jnp.float32),
    mesh=_mesh,
    scratch_types=[
        pltpu.VMEM_SHARED((NP, L), jnp.float32),
        pltpu.VMEM((NCK, CK), jnp.int32),
        pltpu.VMEM((CK, L), jnp.float32),
    ],
    compiler_params=pltpu.CompilerParams(needs_layout_passes=False,
                                         use_tc_tiling_on_sc=False),
)
def _deg_kernel(dst_hbm, zeros_hbm, deg_out, acc_sh, dstb, onesb):
    c = lax.axis_index("c")
    s = lax.axis_index("s")
    wid = c * NS + s

    ones = jnp.ones((L,), jnp.float32)

    def oloop(r, carry):
        onesb[r, :] = ones
        return carry

    lax.fori_loop(0, CK, oloop, 0)

    pltpu.sync_copy(zeros_hbm.at[pl.ds(s * RPW, RPW), :],
                    acc_sh.at[pl.ds(s * RPW, RPW), :])
    pltpu.sync_copy(dst_hbm.at[pl.ds(wid * NCK, NCK), :], dstb)
    plsc.subcore_barrier()

    def eloop(j, carry):
        pltpu.sync_copy(onesb, acc_sh.at[dstb.at[j]], add=True)
        return carry

    lax.fori_loop(0, NCK, eloop, 0)

    plsc.subcore_barrier()
    pltpu.sync_copy(acc_sh.at[pl.ds(s * RPW, RPW), :],
                    deg_out.at[c, pl.ds(s * RPW, RPW), :])


NBUF = 8
NGRP = NCK // NBUF


@functools.partial(
    pl.kernel,
    out_type=jax.ShapeDtypeStruct((NC, NP, DIM), jnp.float32),
    mesh=_mesh,
    scratch_types=[
        pltpu.VMEM_SHARED((NP, DIM), jnp.float32),
        pltpu.VMEM((NCK, CK), jnp.int32),
        pltpu.VMEM((NCK, CK), jnp.int32),
    ] + [pltpu.VMEM((CK, DIM), jnp.float32) for _ in range(NBUF)]
      + [pltpu.SemaphoreType.DMA for _ in range(2 * NBUF)],
    compiler_params=pltpu.CompilerParams(needs_layout_passes=False,
                                         use_tc_tiling_on_sc=False),
)
def _spmm_kernel(z_hbm, src_hbm, dst_hbm, zeros_hbm, w_out,
                 acc_sh, srcb, dstb, *bufs_sems):
    rows = bufs_sems[:NBUF]
    gsem = bufs_sems[NBUF:2 * NBUF]
    ssem = bufs_sems[2 * NBUF:]
    c = lax.axis_index("c")
    s = lax.axis_index("s")
    wid = c * NS + s

    d0 = pltpu.async_copy(zeros_hbm.at[pl.ds(s * RPW, RPW), :],
                          acc_sh.at[pl.ds(s * RPW, RPW), :], gsem[0])
    d1 = pltpu.async_copy(src_hbm.at[pl.ds(wid * NCK, NCK), :], srcb,
                          gsem[1])
    d2 = pltpu.async_copy(dst_hbm.at[pl.ds(wid * NCK, NCK), :], dstb,
                          gsem[2])
    d0.wait()
    d1.wait()
    d2.wait()
    plsc.subcore_barrier()

    def _gather(j, b):
        return pltpu.async_copy(z_hbm.at[srcb.at[j]], rows[b], gsem[b])

    def _scatter(j, b):
        return pltpu.async_copy(rows[b], acc_sh.at[dstb.at[j]], ssem[b],
                                add=True)

    def _gather_wait(j, b):
        pltpu.make_async_copy(z_hbm.at[srcb.at[j]], rows[b], gsem[b]).wait()

    def _scatter_wait(j, b):
        pltpu.make_async_copy(rows[b], acc_sh.at[dstb.at[j]],
                              ssem[b]).wait()

    for b in range(NBUF):
        _gather(b, b)

    def grp_body(grp, carry):
        j0 = grp * NBUF
        for b in range(NBUF):
            _gather_wait(j0 + b, b)
            _scatter(j0 + b, b)
        for b in range(NBUF):
            _scatter_wait(j0 + b, b)
            _gather(j0 + NBUF + b, b)
        return carry

    lax.fori_loop(0, NGRP - 1, grp_body, 0)

    jl = (NGRP - 1) * NBUF
    for b in range(NBUF):
        _gather_wait(jl + b, b)
        _scatter(jl + b, b)
    for b in range(NBUF):
        _scatter_wait(jl + b, b)

    plsc.subcore_barrier()
    pltpu.sync_copy(acc_sh.at[pl.ds(s * RPW, RPW), :],
                    w_out.at[c, pl.ds(s * RPW, RPW), :])


def _stage_a_body(x_ref, w1_ref, degp_ref, z_ref, dinv_ref):
    deg = degp_ref[0, :, 0:1] + degp_ref[1, :, 0:1] + 1.0
    dinv = lax.rsqrt(deg)
    h = jnp.dot(x_ref[...], w1_ref[...],
                preferred_element_type=jnp.float32,
                precision=lax.Precision.HIGHEST)
    z_ref[...] = h * dinv
    dinv_ref[...] = dinv


def _stage_conv_body(relu, w_ref, z_ref, dinv_ref, b_ref, g_ref, bb_ref,
                     wn_ref, zn_ref):
    dinv = dinv_ref[...]
    h = (w_ref[0] + w_ref[1] + z_ref[...]) * dinv + b_ref[...]
    if relu:
        h = jnp.maximum(h, 0.0)
    h = h * (g_ref[...] * CINV) + bb_ref[...]
    zn_ref[...] = jnp.dot(h, wn_ref[...],
                          preferred_element_type=jnp.float32,
                          precision=lax.Precision.HIGHEST) * dinv


def _stage_e_body(w_ref, z_ref, dinv_ref, b_ref, g_ref, bb_ref, batch_ref,
                  mlpW_ref, mlpB_ref, bnmg_ref, bnmb_ref, outW_ref, outb_ref,
                  out_ref, acc_ref):
    i = pl.program_id(0)
    h = (w_ref[0] + w_ref[1] + z_ref[...]) * dinv_ref[...] + b_ref[...]
    h = h * (g_ref[...] * CINV) + bb_ref[...]
    gids = lax.broadcasted_iota(jnp.int32, (1, NG), 1)
    onehot = (batch_ref[...] == gids).astype(jnp.float32)
    contrib = lax.dot_general(onehot, h, (((0,), (0,)), ((), ())),
                              preferred_element_type=jnp.float32,
                              precision=lax.Precision.HIGHEST)

    @pl.when(i == 0)
    def _():
        acc_ref[...] = contrib

    @pl.when(i > 0)
    def _():
        acc_ref[...] = acc_ref[...] + contrib

    @pl.when(i == G - 1)
    def _():
        p = acc_ref[...]
        for k in range(NHID):
            p = jnp.dot(p, mlpW_ref[k],
                        preferred_element_type=jnp.float32,
                        precision=lax.Precision.HIGHEST) + mlpB_ref[k]
            p = jnp.maximum(p, 0.0)
            p = p * (bnmg_ref[k] * CINV) + bnmb_ref[k]
        out_ref[...] = jnp.dot(p, outW_ref[...],
                               preferred_element_type=jnp.float32,
                               precision=lax.Precision.HIGHEST) + outb_ref[...]


def _full_spec(shape):
    return pl.BlockSpec(shape, lambda i: tuple(0 for _ in shape))


_stage_a = pl.pallas_call(
    _stage_a_body,
    grid=(G,),
    in_specs=[
        pl.BlockSpec((BLK, F_IN), lambda i: (i, 0)),
        _full_spec((F_IN, DIM)),
        pl.BlockSpec((NC, BLK, L), lambda i: (0, i, 0)),
    ],
    out_specs=[
        pl.BlockSpec((BLK, DIM), lambda i: (i, 0)),
        pl.BlockSpec((BLK, 1), lambda i: (i, 0)),
    ],
    out_shape=[
        jax.ShapeDtypeStruct((NP, DIM), jnp.float32),
        jax.ShapeDtypeStruct((NP, 1), jnp.float32),
    ],
)


def _make_stage_conv(relu):
    return pl.pallas_call(
        functools.partial(_stage_conv_body, relu),
        grid=(G,),
        in_specs=[
            pl.BlockSpec((NC, BLK, DIM), lambda i: (0, i, 0)),
            pl.BlockSpec((BLK, DIM), lambda i: (i, 0)),
            pl.BlockSpec((BLK, 1), lambda i: (i, 0)),
            _full_spec((1, DIM)),
            _full_spec((1, DIM)),
            _full_spec((1, DIM)),
            _full_spec((DIM, DIM)),
        ],
        out_specs=pl.BlockSpec((BLK, DIM), lambda i: (i, 0)),
        out_shape=jax.ShapeDtypeStruct((NP, DIM), jnp.float32),
    )


_stage_conv_relu = _make_stage_conv(True)
_stage_conv_plain = _make_stage_conv(False)

_stage_e = pl.pallas_call(
    _stage_e_body,
    grid=(G,),
    in_specs=[
        pl.BlockSpec((NC, BLK, DIM), lambda i: (0, i, 0)),
        pl.BlockSpec((BLK, DIM), lambda i: (i, 0)),
        pl.BlockSpec((BLK, 1), lambda i: (i, 0)),
        _full_spec((1, DIM)),
        _full_spec((1, DIM)),
        _full_spec((1, DIM)),
        pl.BlockSpec((BLK, 1), lambda i: (i, 0)),
        _full_spec((NHID, DIM, DIM)),
        _full_spec((NHID, DIM)),
        _full_spec((NHID, DIM)),
        _full_spec((NHID, DIM)),
        _full_spec((DIM, 1)),
        _full_spec((1, 1)),
    ],
    out_specs=pl.BlockSpec((NG, 1), lambda i: (0, 0)),
    out_shape=jax.ShapeDtypeStruct((NG, 1), jnp.float32),
    scratch_shapes=[pltpu.VMEM((NG, DIM), jnp.float32)],
)


def kernel(x, edge_index, batch, W1, b1, bn1_g, bn1_b, convW, convB,
           bnc_g, bnc_b, mlpW, mlpB, bnm_g, bnm_b, outW, outb):
    src = edge_index[0]
    dst = edge_index[1]
    pad = N + jnp.arange(EP - E, dtype=jnp.int32) % (NP - N)
    src2d = jnp.concatenate([src, pad]).reshape(EP // CK, CK)
    dst2d = jnp.concatenate([dst, pad]).reshape(EP // CK, CK)
    xp = x
    batchp = jnp.concatenate(
        [batch, jnp.full((NP - N,), NG, jnp.int32)]).reshape(NP, 1)
    zeros_nd = jnp.zeros((NP, DIM), jnp.float32)
    zeros_nl = jnp.zeros((NP, L), jnp.float32)

    b1r = b1.reshape(1, DIM)
    g1r = bn1_g.reshape(1, DIM)
    bb1r = bn1_b.reshape(1, DIM)
    outbr = outb.reshape(1, 1)

    deg_parts = _deg_kernel(dst2d, zeros_nl)
    z, dinv = _stage_a(xp, W1, deg_parts)

    stage_params = [(b1r, g1r, bb1r, True)] + [
        (convB[i].reshape(1, DIM), bnc_g[i].reshape(1, DIM),
         bnc_b[i].reshape(1, DIM), False)
        for i in range(NHID)
    ]

    for li in range(NHID):
        w = _spmm_kernel(z, src2d, dst2d, zeros_nd)
        br, gr, bbr, relu = stage_params[li]
        stage = _stage_conv_relu if relu else _stage_conv_plain
        z = stage(w, z, dinv, br, gr, bbr, convW[li])

    w = _spmm_kernel(z, src2d, dst2d, zeros_nd)
    br, gr, bbr, _ = stage_params[NHID]
    out = _stage_e(w, z, dinv, br, gr, bbr, batchp,
                   mlpW, mlpB, bnm_g, bnm_b, outW, outbr)
    return out


# speedup vs baseline: 36.9457x; 1.0021x over previous
import functools
import math

import jax
import jax.numpy as jnp
from jax import lax
from jax.experimental import pallas as pl
from jax.experimental.pallas import tpu as pltpu
from jax.experimental.pallas import tpu_sc as plsc

N = 10000
E = 320000
F_IN = 128
DIM = 64
NG = 64
NHID = 3
EPS = 1e-5

NC = 2
NS = 16
NW = NC * NS
L = 16

NP = 10240
EP = 327680
CK = 128
EW = EP // NW
NCK = EW // CK
RPW = NP // NS
PAD_ROW = 10008

BLK = 2048
G = NP // BLK

CINV = 1.0 / math.sqrt(1.0 + EPS)

_mesh = plsc.VectorSubcoreMesh(core_axis_name="c", subcore_axis_name="s")


@functools.partial(
    pl.kernel,
    out_type=jax.ShapeDtypeStruct((NC, NP, L), jnp.float32),
    mesh=_mesh,
    scratch_types=[
        pltpu.VMEM_SHARED((NP, L), jnp.float32),
        pltpu.VMEM((NCK, CK), jnp.int32),
        pltpu.VMEM((CK, L), jnp.float32),
    ],
    compiler_params=pltpu.CompilerParams(needs_layout_passes=False,
                                         use_tc_tiling_on_sc=False),
)
def _deg_kernel(dst_hbm, zeros_hbm, deg_out, acc_sh, dstb, onesb):
    c = lax.axis_index("c")
    s = lax.axis_index("s")
    wid = c * NS + s

    ones = jnp.ones((L,), jnp.float32)

    def oloop(r, carry):
        onesb[r, :] = ones
        return carry

    lax.fori_loop(0, CK, oloop, 0)

    pltpu.sync_copy(zeros_hbm.at[pl.ds(s * RPW, RPW), :],
                    acc_sh.at[pl.ds(s * RPW, RPW), :])
    pltpu.sync_copy(dst_hbm.at[pl.ds(wid * NCK, NCK), :], dstb)
    plsc.subcore_barrier()

    def eloop(j, carry):
        pltpu.sync_copy(onesb, acc_sh.at[dstb.at[j]], add=True)
        return carry

    lax.fori_loop(0, NCK, eloop, 0)

    plsc.subcore_barrier()
    pltpu.sync_copy(acc_sh.at[pl.ds(s * RPW, RPW), :],
                    deg_out.at[c, pl.ds(s * RPW, RPW), :])


NBUF = 8
NGRP = NCK // NBUF


@functools.partial(
    pl.kernel,
    out_type=jax.ShapeDtypeStruct((NC, NP, DIM), jnp.float32),
    mesh=_mesh,
    scratch_types=[
        pltpu.VMEM_SHARED((NP, DIM), jnp.float32),
        pltpu.VMEM((NCK, CK), jnp.int32),
        pltpu.VMEM((NCK, CK), jnp.int32),
    ] + [pltpu.VMEM((CK, DIM), jnp.float32) for _ in range(NBUF)]
      + [pltpu.SemaphoreType.DMA for _ in range(2 * NBUF)],
    compiler_params=pltpu.CompilerParams(needs_layout_passes=False,
                                         use_tc_tiling_on_sc=False),
)
def _spmm_kernel(z_hbm, src_hbm, dst_hbm, zeros_hbm, w_out,
                 acc_sh, srcb, dstb, *bufs_sems):
    rows = bufs_sems[:NBUF]
    gsem = bufs_sems[NBUF:2 * NBUF]
    ssem = bufs_sems[2 * NBUF:]
    c = lax.axis_index("c")
    s = lax.axis_index("s")
    wid = c * NS + s

    d0 = pltpu.async_copy(zeros_hbm.at[pl.ds(s * RPW, RPW), :],
                          acc_sh.at[pl.ds(s * RPW, RPW), :], gsem[0])
    d1 = pltpu.async_copy(src_hbm.at[pl.ds(wid * NCK, NCK), :], srcb,
                          gsem[1])
    d2 = pltpu.async_copy(dst_hbm.at[pl.ds(wid * NCK, NCK), :], dstb,
                          gsem[2])
    d0.wait()
    d1.wait()
    d2.wait()
    plsc.subcore_barrier()

    def _gather(j, b):
        return pltpu.async_copy(z_hbm.at[srcb.at[j]], rows[b], gsem[b])

    def _scatter(j, b):
        return pltpu.async_copy(rows[b], acc_sh.at[dstb.at[j]], ssem[b],
                                add=True)

    def _gather_wait(j, b):
        pltpu.make_async_copy(z_hbm.at[srcb.at[j]], rows[b], gsem[b]).wait()

    def _scatter_wait(j, b):
        pltpu.make_async_copy(rows[b], acc_sh.at[dstb.at[j]],
                              ssem[b]).wait()

    for b in range(NBUF):
        _gather(b, b)

    def grp_body(grp, carry):
        j0 = grp * NBUF
        for b in range(NBUF):
            _gather_wait(j0 + b, b)
            _scatter(j0 + b, b)
        for b in range(NBUF):
            _scatter_wait(j0 + b, b)
            _gather(j0 + NBUF + b, b)
        return carry

    lax.fori_loop(0, NGRP - 1, grp_body, 0)

    jl = (NGRP - 1) * NBUF
    for b in range(NBUF):
        _gather_wait(jl + b, b)
        _scatter(jl + b, b)
    for b in range(NBUF):
        _scatter_wait(jl + b, b)

    plsc.subcore_barrier()
    pltpu.sync_copy(acc_sh.at[pl.ds(s * RPW, RPW), :],
                    w_out.at[c, pl.ds(s * RPW, RPW), :])


def _matmul_a_body(x_ref, w1_ref, y_ref):
    y_ref[...] = jnp.dot(x_ref[...], w1_ref[...],
                         preferred_element_type=jnp.float32,
                         precision=lax.Precision.HIGHEST)


def _scale_a_body(y_ref, degp_ref, z_ref, dinv_ref):
    deg = degp_ref[0, :, 0:1] + degp_ref[1, :, 0:1] + 1.0
    dinv = lax.rsqrt(deg)
    z_ref[...] = y_ref[...] * dinv
    dinv_ref[...] = dinv


def _stage_conv_body(relu, w_ref, z_ref, dinv_ref, b_ref, g_ref, bb_ref,
                     wn_ref, zn_ref):
    dinv = dinv_ref[...]
    h = (w_ref[0] + w_ref[1] + z_ref[...]) * dinv + b_ref[...]
    if relu:
        h = jnp.maximum(h, 0.0)
    h = h * (g_ref[...] * CINV) + bb_ref[...]
    zn_ref[...] = jnp.dot(h, wn_ref[...],
                          preferred_element_type=jnp.float32,
                          precision=lax.Precision.HIGHEST) * dinv


def _stage_e_body(w_ref, z_ref, dinv_ref, b_ref, g_ref, bb_ref, batch_ref,
                  mlpW_ref, mlpB_ref, bnmg_ref, bnmb_ref, outW_ref, outb_ref,
                  out_ref, acc_ref):
    i = pl.program_id(0)
    h = (w_ref[0] + w_ref[1] + z_ref[...]) * dinv_ref[...] + b_ref[...]
    h = h * (g_ref[...] * CINV) + bb_ref[...]
    gids = lax.broadcasted_iota(jnp.int32, (1, NG), 1)
    onehot = (batch_ref[...] == gids).astype(jnp.float32)
    contrib = lax.dot_general(onehot, h, (((0,), (0,)), ((), ())),
                              preferred_element_type=jnp.float32,
                              precision=lax.Precision.HIGHEST)

    @pl.when(i == 0)
    def _():
        acc_ref[...] = contrib

    @pl.when(i > 0)
    def _():
        acc_ref[...] = acc_ref[...] + contrib

    @pl.when(i == G - 1)
    def _():
        p = acc_ref[...]
        for k in range(NHID):
            p = jnp.dot(p, mlpW_ref[k],
                        preferred_element_type=jnp.float32,
                        precision=lax.Precision.HIGHEST) + mlpB_ref[k]
            p = jnp.maximum(p, 0.0)
            p = p * (bnmg_ref[k] * CINV) + bnmb_ref[k]
        out_ref[...] = jnp.dot(p, outW_ref[...],
                               preferred_element_type=jnp.float32,
                               precision=lax.Precision.HIGHEST) + outb_ref[...]


def _full_spec(shape):
    return pl.BlockSpec(shape, lambda i: tuple(0 for _ in shape))


_matmul_a = pl.pallas_call(
    _matmul_a_body,
    grid=(G,),
    in_specs=[
        pl.BlockSpec((BLK, F_IN), lambda i: (i, 0)),
        _full_spec((F_IN, DIM)),
    ],
    out_specs=pl.BlockSpec((BLK, DIM), lambda i: (i, 0)),
    out_shape=jax.ShapeDtypeStruct((NP, DIM), jnp.float32),
)

_scale_a = pl.pallas_call(
    _scale_a_body,
    grid=(G,),
    in_specs=[
        pl.BlockSpec((BLK, DIM), lambda i: (i, 0)),
        pl.BlockSpec((NC, BLK, L), lambda i: (0, i, 0)),
    ],
    out_specs=[
        pl.BlockSpec((BLK, DIM), lambda i: (i, 0)),
        pl.BlockSpec((BLK, 1), lambda i: (i, 0)),
    ],
    out_shape=[
        jax.ShapeDtypeStruct((NP, DIM), jnp.float32),
        jax.ShapeDtypeStruct((NP, 1), jnp.float32),
    ],
)


def _make_stage_conv(relu):
    return pl.pallas_call(
        functools.partial(_stage_conv_body, relu),
        grid=(G,),
        in_specs=[
            pl.BlockSpec((NC, BLK, DIM), lambda i: (0, i, 0)),
            pl.BlockSpec((BLK, DIM), lambda i: (i, 0)),
            pl.BlockSpec((BLK, 1), lambda i: (i, 0)),
            _full_spec((1, DIM)),
            _full_spec((1, DIM)),
            _full_spec((1, DIM)),
            _full_spec((DIM, DIM)),
        ],
        out_specs=pl.BlockSpec((BLK, DIM), lambda i: (i, 0)),
        out_shape=jax.ShapeDtypeStruct((NP, DIM), jnp.float32),
    )


_stage_conv_relu = _make_stage_conv(True)
_stage_conv_plain = _make_stage_conv(False)

_stage_e = pl.pallas_call(
    _stage_e_body,
    grid=(G,),
    in_specs=[
        pl.BlockSpec((NC, BLK, DIM), lambda i: (0, i, 0)),
        pl.BlockSpec((BLK, DIM), lambda i: (i, 0)),
        pl.BlockSpec((BLK, 1), lambda i: (i, 0)),
        _full_spec((1, DIM)),
        _full_spec((1, DIM)),
        _full_spec((1, DIM)),
        pl.BlockSpec((BLK, 1), lambda i: (i, 0)),
        _full_spec((NHID, DIM, DIM)),
        _full_spec((NHID, DIM)),
        _full_spec((NHID, DIM)),
        _full_spec((NHID, DIM)),
        _full_spec((DIM, 1)),
        _full_spec((1, 1)),
    ],
    out_specs=pl.BlockSpec((NG, 1), lambda i: (0, 0)),
    out_shape=jax.ShapeDtypeStruct((NG, 1), jnp.float32),
    scratch_shapes=[pltpu.VMEM((NG, DIM), jnp.float32)],
)


def kernel(x, edge_index, batch, W1, b1, bn1_g, bn1_b, convW, convB,
           bnc_g, bnc_b, mlpW, mlpB, bnm_g, bnm_b, outW, outb):
    src = edge_index[0]
    dst = edge_index[1]
    pad = N + jnp.arange(EP - E, dtype=jnp.int32) % (NP - N)
    src2d = jnp.concatenate([src, pad]).reshape(EP // CK, CK)
    dst2d = jnp.concatenate([dst, pad]).reshape(EP // CK, CK)
    xp = x
    batchp = jnp.concatenate(
        [batch, jnp.full((NP - N,), NG, jnp.int32)]).reshape(NP, 1)
    zeros_nd = jnp.zeros((NP, DIM), jnp.float32)
    zeros_nl = jnp.zeros((NP, L), jnp.float32)

    b1r = b1.reshape(1, DIM)
    g1r = bn1_g.reshape(1, DIM)
    bb1r = bn1_b.reshape(1, DIM)
    outbr = outb.reshape(1, 1)

    y1 = _matmul_a(xp, W1)
    deg_parts = _deg_kernel(dst2d, zeros_nl)
    z, dinv = _scale_a(y1, deg_parts)

    stage_params = [(b1r, g1r, bb1r, True)] + [
        (convB[i].reshape(1, DIM), bnc_g[i].reshape(1, DIM),
         bnc_b[i].reshape(1, DIM), False)
        for i in range(NHID)
    ]

    for li in range(NHID):
        w = _spmm_kernel(z, src2d, dst2d, zeros_nd)
        br, gr, bbr, relu = stage_params[li]
        stage = _stage_conv_relu if relu else _stage_conv_plain
        z = stage(w, z, dinv, br, gr, bbr, convW[li])

    w = _spmm_kernel(z, src2d, dst2d, zeros_nd)
    br, gr, bbr, _ = stage_params[NHID]
    out = _stage_e(w, z, dinv, br, gr, bbr, batchp,
                   mlpW, mlpB, bnm_g, bnm_b, outW, outbr)
    return out


# speedup vs baseline: 37.7599x; 1.0220x over previous
import functools
import math

import jax
import jax.numpy as jnp
from jax import lax
from jax.experimental import pallas as pl
from jax.experimental.pallas import tpu as pltpu
from jax.experimental.pallas import tpu_sc as plsc

N = 10000
E = 320000
F_IN = 128
DIM = 64
NG = 64
NHID = 3
EPS = 1e-5

NC = 2
NS = 16
NW = NC * NS
L = 16

NP = 10240
EP = 327680
CK = 128
EW = EP // NW
NCK = EW // CK
RPW = NP // NS
PAD_ROW = 10008

BLK = 2048
G = NP // BLK

CINV = 1.0 / math.sqrt(1.0 + EPS)

_mesh = plsc.VectorSubcoreMesh(core_axis_name="c", subcore_axis_name="s")


@functools.partial(
    pl.kernel,
    out_type=jax.ShapeDtypeStruct((NC, NP, L), jnp.float32),
    mesh=_mesh,
    scratch_types=[
        pltpu.VMEM_SHARED((NP, L), jnp.float32),
        pltpu.VMEM((NCK, CK), jnp.int32),
        pltpu.VMEM((CK, L), jnp.float32),
    ],
    compiler_params=pltpu.CompilerParams(needs_layout_passes=False,
                                         use_tc_tiling_on_sc=False),
)
def _deg_kernel(dst_hbm, zeros_hbm, deg_out, acc_sh, dstb, onesb):
    c = lax.axis_index("c")
    s = lax.axis_index("s")
    wid = c * NS + s

    ones = jnp.ones((L,), jnp.float32)

    def oloop(r, carry):
        onesb[r, :] = ones
        return carry

    lax.fori_loop(0, CK, oloop, 0)

    pltpu.sync_copy(zeros_hbm.at[pl.ds(s * RPW, RPW), :],
                    acc_sh.at[pl.ds(s * RPW, RPW), :])
    pltpu.sync_copy(dst_hbm.at[pl.ds(wid * NCK, NCK), :], dstb)
    plsc.subcore_barrier()

    def eloop(j, carry):
        pltpu.sync_copy(onesb, acc_sh.at[dstb.at[j]], add=True)
        return carry

    lax.fori_loop(0, NCK, eloop, 0)

    plsc.subcore_barrier()
    pltpu.sync_copy(acc_sh.at[pl.ds(s * RPW, RPW), :],
                    deg_out.at[c, pl.ds(s * RPW, RPW), :])


NBUF = 8
NGRP = NCK // NBUF


@functools.partial(
    pl.kernel,
    out_type=jax.ShapeDtypeStruct((NC, NP, DIM), jnp.float32),
    mesh=_mesh,
    scratch_types=[
        pltpu.VMEM_SHARED((NP, DIM), jnp.float32),
        pltpu.VMEM((NCK, CK), jnp.int32),
        pltpu.VMEM((NCK, CK), jnp.int32),
    ] + [pltpu.VMEM((CK, DIM), jnp.float32) for _ in range(NBUF)]
      + [pltpu.SemaphoreType.DMA for _ in range(2 * NBUF)],
    compiler_params=pltpu.CompilerParams(needs_layout_passes=False,
                                         use_tc_tiling_on_sc=False),
)
def _spmm_kernel(z_hbm, src_hbm, dst_hbm, w_out,
                 acc_sh, srcb, dstb, *bufs_sems):
    rows = bufs_sems[:NBUF]
    gsem = bufs_sems[NBUF:2 * NBUF]
    ssem = bufs_sems[2 * NBUF:]
    c = lax.axis_index("c")
    s = lax.axis_index("s")
    wid = c * NS + s

    d1 = pltpu.async_copy(src_hbm.at[pl.ds(wid * NCK, NCK), :], srcb,
                          gsem[6])
    d2 = pltpu.async_copy(dst_hbm.at[pl.ds(wid * NCK, NCK), :], dstb,
                          gsem[7])
    zvec = jnp.zeros((L,), jnp.float32)

    def zloop(r, carry):
        for u in range(DIM // L):
            rows[0][r, pl.ds(u * L, L)] = zvec
        return carry

    lax.fori_loop(0, CK, zloop, 0)
    zcopies = [
        pltpu.async_copy(rows[0],
                         acc_sh.at[pl.ds(s * RPW + k * CK, CK), :],
                         gsem[k])
        for k in range(RPW // CK)
    ]
    for d in zcopies:
        d.wait()
    d1.wait()
    d2.wait()
    plsc.subcore_barrier()

    def _gather(j, b):
        return pltpu.async_copy(z_hbm.at[srcb.at[j]], rows[b], gsem[b])

    def _scatter(j, b):
        return pltpu.async_copy(rows[b], acc_sh.at[dstb.at[j]], ssem[b],
                                add=True)

    def _gather_wait(j, b):
        pltpu.make_async_copy(z_hbm.at[srcb.at[j]], rows[b], gsem[b]).wait()

    def _scatter_wait(j, b):
        pltpu.make_async_copy(rows[b], acc_sh.at[dstb.at[j]],
                              ssem[b]).wait()

    for b in range(NBUF):
        _gather(b, b)

    def grp_body(grp, carry):
        j0 = grp * NBUF
        for b in range(NBUF):
            _gather_wait(j0 + b, b)
            _scatter(j0 + b, b)
        for b in range(NBUF):
            _scatter_wait(j0 + b, b)
            _gather(j0 + NBUF + b, b)
        return carry

    lax.fori_loop(0, NGRP - 1, grp_body, 0)

    jl = (NGRP - 1) * NBUF
    for b in range(NBUF):
        _gather_wait(jl + b, b)
        _scatter(jl + b, b)
    for b in range(NBUF):
        _scatter_wait(jl + b, b)

    plsc.subcore_barrier()
    pltpu.sync_copy(acc_sh.at[pl.ds(s * RPW, RPW), :],
                    w_out.at[c, pl.ds(s * RPW, RPW), :])


def _matmul_a_body(x_ref, w1_ref, y_ref):
    y_ref[...] = jnp.dot(x_ref[...], w1_ref[...],
                         preferred_element_type=jnp.float32,
                         precision=lax.Precision.HIGHEST)


def _scale_a_body(y_ref, degp_ref, z_ref, dinv_ref):
    deg = degp_ref[0, :, 0:1] + degp_ref[1, :, 0:1] + 1.0
    dinv = lax.rsqrt(deg)
    z_ref[...] = y_ref[...] * dinv
    dinv_ref[...] = dinv


def _stage_conv_body(relu, w_ref, z_ref, dinv_ref, b_ref, g_ref, bb_ref,
                     wn_ref, zn_ref):
    dinv = dinv_ref[...]
    h = (w_ref[0] + w_ref[1] + z_ref[...]) * dinv + b_ref[...]
    if relu:
        h = jnp.maximum(h, 0.0)
    h = h * (g_ref[...] * CINV) + bb_ref[...]
    zn_ref[...] = jnp.dot(h, wn_ref[...],
                          preferred_element_type=jnp.float32,
                          precision=lax.Precision.HIGHEST) * dinv


def _stage_e_body(w_ref, z_ref, dinv_ref, b_ref, g_ref, bb_ref, batch_ref,
                  mlpW_ref, mlpB_ref, bnmg_ref, bnmb_ref, outW_ref, outb_ref,
                  out_ref, acc_ref):
    i = pl.program_id(0)
    h = (w_ref[0] + w_ref[1] + z_ref[...]) * dinv_ref[...] + b_ref[...]
    h = h * (g_ref[...] * CINV) + bb_ref[...]
    gids = lax.broadcasted_iota(jnp.int32, (1, NG), 1)
    onehot = (batch_ref[...] == gids).astype(jnp.float32)
    contrib = lax.dot_general(onehot, h, (((0,), (0,)), ((), ())),
                              preferred_element_type=jnp.float32,
                              precision=lax.Precision.HIGHEST)

    @pl.when(i == 0)
    def _():
        acc_ref[...] = contrib

    @pl.when(i > 0)
    def _():
        acc_ref[...] = acc_ref[...] + contrib

    @pl.when(i == G - 1)
    def _():
        p = acc_ref[...]
        for k in range(NHID):
            p = jnp.dot(p, mlpW_ref[k],
                        preferred_element_type=jnp.float32,
                        precision=lax.Precision.HIGHEST) + mlpB_ref[k]
            p = jnp.maximum(p, 0.0)
            p = p * (bnmg_ref[k] * CINV) + bnmb_ref[k]
        out_ref[...] = jnp.dot(p, outW_ref[...],
                               preferred_element_type=jnp.float32,
                               precision=lax.Precision.HIGHEST) + outb_ref[...]


def _full_spec(shape):
    return pl.BlockSpec(shape, lambda i: tuple(0 for _ in shape))


_matmul_a = pl.pallas_call(
    _matmul_a_body,
    grid=(G,),
    in_specs=[
        pl.BlockSpec((BLK, F_IN), lambda i: (i, 0)),
        _full_spec((F_IN, DIM)),
    ],
    out_specs=pl.BlockSpec((BLK, DIM), lambda i: (i, 0)),
    out_shape=jax.ShapeDtypeStruct((NP, DIM), jnp.float32),
)

_scale_a = pl.pallas_call(
    _scale_a_body,
    grid=(G,),
    in_specs=[
        pl.BlockSpec((BLK, DIM), lambda i: (i, 0)),
        pl.BlockSpec((NC, BLK, L), lambda i: (0, i, 0)),
    ],
    out_specs=[
        pl.BlockSpec((BLK, DIM), lambda i: (i, 0)),
        pl.BlockSpec((BLK, 1), lambda i: (i, 0)),
    ],
    out_shape=[
        jax.ShapeDtypeStruct((NP, DIM), jnp.float32),
        jax.ShapeDtypeStruct((NP, 1), jnp.float32),
    ],
)


def _make_stage_conv(relu):
    return pl.pallas_call(
        functools.partial(_stage_conv_body, relu),
        grid=(G,),
        in_specs=[
            pl.BlockSpec((NC, BLK, DIM), lambda i: (0, i, 0)),
            pl.BlockSpec((BLK, DIM), lambda i: (i, 0)),
            pl.BlockSpec((BLK, 1), lambda i: (i, 0)),
            _full_spec((1, DIM)),
            _full_spec((1, DIM)),
            _full_spec((1, DIM)),
            _full_spec((DIM, DIM)),
        ],
        out_specs=pl.BlockSpec((BLK, DIM), lambda i: (i, 0)),
        out_shape=jax.ShapeDtypeStruct((NP, DIM), jnp.float32),
    )


_stage_conv_relu = _make_stage_conv(True)
_stage_conv_plain = _make_stage_conv(False)

_stage_e = pl.pallas_call(
    _stage_e_body,
    grid=(G,),
    in_specs=[
        pl.BlockSpec((NC, BLK, DIM), lambda i: (0, i, 0)),
        pl.BlockSpec((BLK, DIM), lambda i: (i, 0)),
        pl.BlockSpec((BLK, 1), lambda i: (i, 0)),
        _full_spec((1, DIM)),
        _full_spec((1, DIM)),
        _full_spec((1, DIM)),
        pl.BlockSpec((BLK, 1), lambda i: (i, 0)),
        _full_spec((NHID, DIM, DIM)),
        _full_spec((NHID, DIM)),
        _full_spec((NHID, DIM)),
        _full_spec((NHID, DIM)),
        _full_spec((DIM, 1)),
        _full_spec((1, 1)),
    ],
    out_specs=pl.BlockSpec((NG, 1), lambda i: (0, 0)),
    out_shape=jax.ShapeDtypeStruct((NG, 1), jnp.float32),
    scratch_shapes=[pltpu.VMEM((NG, DIM), jnp.float32)],
)


def kernel(x, edge_index, batch, W1, b1, bn1_g, bn1_b, convW, convB,
           bnc_g, bnc_b, mlpW, mlpB, bnm_g, bnm_b, outW, outb):
    src = edge_index[0]
    dst = edge_index[1]
    pad = N + jnp.arange(EP - E, dtype=jnp.int32) % (NP - N)
    src2d = jnp.concatenate([src, pad]).reshape(EP // CK, CK)
    dst2d = jnp.concatenate([dst, pad]).reshape(EP // CK, CK)
    xp = x
    batchp = jnp.concatenate(
        [batch, jnp.full((NP - N,), NG, jnp.int32)]).reshape(NP, 1)
    zeros_nl = jnp.zeros((NP, L), jnp.float32)

    b1r = b1.reshape(1, DIM)
    g1r = bn1_g.reshape(1, DIM)
    bb1r = bn1_b.reshape(1, DIM)
    outbr = outb.reshape(1, 1)

    y1 = _matmul_a(xp, W1)
    deg_parts = _deg_kernel(dst2d, zeros_nl)
    z, dinv = _scale_a(y1, deg_parts)

    stage_params = [(b1r, g1r, bb1r, True)] + [
        (convB[i].reshape(1, DIM), bnc_g[i].reshape(1, DIM),
         bnc_b[i].reshape(1, DIM), False)
        for i in range(NHID)
    ]

    for li in range(NHID):
        w = _spmm_kernel(z, src2d, dst2d)
        br, gr, bbr, relu = stage_params[li]
        stage = _stage_conv_relu if relu else _stage_conv_plain
        z = stage(w, z, dinv, br, gr, bbr, convW[li])

    w = _spmm_kernel(z, src2d, dst2d)
    br, gr, bbr, _ = stage_params[NHID]
    out = _stage_e(w, z, dinv, br, gr, bbr, batchp,
                   mlpW, mlpB, bnm_g, bnm_b, outW, outbr)
    return out
